# baseline (speedup 1.0000x reference)
import functools

import jax
import jax.numpy as jnp
from jax import lax
from jax.experimental import pallas as pl
from jax.experimental.pallas import tpu as pltpu
from jax.experimental.pallas import tpu_sc as plsc

N = 10000
E = 160000
H = 128
R = 2
L = 2
K = 32
EPS = 1e-7

NW = 32
NP = 20480
G = 4
CE = 40
HG = H // 16

RT0, RT1 = 1280, 0
C0ROWS = 16 * RT0
ET0, ET1 = 10000, 10000
C0E = 16 * ET0
ST0, ST1 = 20480, 20480
C0S = 16 * ST0

IPAD = 40960

_MESH = plsc.VectorSubcoreMesh(core_axis_name="c", subcore_axis_name="s")
_SC_PARAMS = pltpu.CompilerParams(needs_layout_passes=False)

NB_BLK = 1000
NBLKS = N // NB_BLK


def _mrow(gb, row):
    return [gb[row, pl.ds(16 * g, 16)] for g in range(HG)]


def _span(t0, t1, c0total):
    c = lax.axis_index("c")
    s = lax.axis_index("s")
    cnt = pl.multiple_of(jnp.where(c == 0, t0, t1), 8)
    base = pl.multiple_of(jnp.where(c == 0, s * t0, c0total + s * t1), 8)
    return cnt, base


@functools.partial(
    pl.kernel, mesh=_MESH,
    out_type=jax.ShapeDtypeStruct((NP * K + IPAD,), jnp.int32),
    scratch_types=[pltpu.VMEM((128,), jnp.int32),
                   pltpu.VMEM((128,), jnp.int32)])
def _k0(nbr_hbm, dst_hbm, d2_hbm, idx_v, d2_v):
    cnt, base = _span(ST0, ST1, C0S)

    @pl.loop(0, cnt, step=128)
    def _(c):
        pltpu.sync_copy(nbr_hbm.at[pl.ds(base + c, 128)], idx_v)
        pltpu.sync_copy(dst_hbm.at[idx_v], d2_v)
        pltpu.sync_copy(d2_v, d2_hbm.at[pl.ds(base + c, 128)])


@functools.partial(
    pl.kernel, mesh=_MESH, compiler_params=_SC_PARAMS,
    out_type=jax.ShapeDtypeStruct((R * E, H), jnp.float32),
    scratch_types=[pltpu.VMEM((ET0,), jnp.int32),
                   pltpu.VMEM((ET0,), jnp.float32),
                   pltpu.VMEM((R, H), jnp.float32),
                   pltpu.VMEM((CE, H), jnp.float32),
                   pltpu.VMEM((CE, H), jnp.float32),
                   pltpu.VMEM((CE, H), jnp.float32),
                   pltpu.VMEM((CE, H), jnp.float32),
                   pltpu.SemaphoreType.DMA,
                   pltpu.SemaphoreType.DMA,
                   pltpu.SemaphoreType.DMA,
                   pltpu.SemaphoreType.DMA])
def _k1(x_hbm, src_hbm, ea_hbm, we_hbm, msg_hbm,
        ia, ab, wev, xbA, xbB, mbA, mbB, gsA, gsB, ssA, ssB):
    ept, base = _span(ET0, ET1, C0E)
    pltpu.sync_copy(we_hbm, wev)
    pltpu.sync_copy(src_hbm.at[pl.ds(base, ET0)], ia)
    pltpu.sync_copy(ea_hbm.at[pl.ds(base, ET0)], ab)
    we0 = [wev[0, pl.ds(16 * g, 16)] for g in range(HG)]
    we1 = [wev[1, pl.ds(16 * g, 16)] for g in range(HG)]

    def gx(c, xb, sem):
        return pltpu.make_async_copy(x_hbm.at[ia.at[pl.ds(c, CE)]], xb, sem)

    def st(c, mb, sem):
        return pltpu.make_async_copy(mb, msg_hbm.at[pl.ds(base + c, CE)], sem)

    def compute(c, xb, mb):
        rk = (base + c) >= E
        wegs = [jnp.where(rk, we1[g], we0[g]) for g in range(HG)]

        @pl.loop(0, CE)
        def _(j):
            a = plsc.load_gather(ab, [jnp.full((16,), c + j, jnp.int32)])
            ms = [jnp.maximum(xb[j, pl.ds(16 * g, 16)] + a * wegs[g], 0.0)
                  + EPS for g in range(HG)]
            for g in range(HG):
                mb[j, pl.ds(16 * g, 16)] = ms[g]

    gx(0, xbA, gsA).start()

    @pl.loop(0, ept, step=2 * CE)
    def _(c):
        gx(c + CE, xbB, gsB).start()
        gx(c, xbA, gsA).wait()

        @pl.when(c >= 2 * CE)
        def _():
            st(c - 2 * CE, mbA, ssA).wait()

        compute(c, xbA, mbA)
        st(c, mbA, ssA).start()

        @pl.when(c + 2 * CE < ept)
        def _():
            gx(c + 2 * CE, xbA, gsA).start()

        gx(c + CE, xbB, gsB).wait()

        @pl.when(c >= 2 * CE)
        def _():
            st(c - CE, mbB, ssB).wait()

        compute(c + CE, xbB, mbB)
        st(c + CE, mbB, ssB).start()

    st(ept - 2 * CE, mbA, ssA).wait()
    st(ept - CE, mbB, ssB).wait()


@functools.partial(
    pl.kernel, mesh=_MESH, compiler_params=_SC_PARAMS,
    out_type=jax.ShapeDtypeStruct((NP, H), jnp.float32),
    scratch_types=[pltpu.VMEM((RT0 * K,), jnp.int32),
                   pltpu.VMEM((G * K, H), jnp.float32),
                   pltpu.VMEM((G * K, H), jnp.float32),
                   pltpu.VMEM((G, H), jnp.float32),
                   pltpu.VMEM((G, H), jnp.float32),
                   pltpu.SemaphoreType.DMA,
                   pltpu.SemaphoreType.DMA,
                   pltpu.SemaphoreType.DMA,
                   pltpu.SemaphoreType.DMA])
def _k2(msg_hbm, nbr_hbm, smax_hbm,
        ia, gbA, gbB, obA, obB, gsA, gsB, ssA, ssB):
    nrows, nbrow = _span(RT0, RT1, C0ROWS)

    @pl.when(nrows > 0)
    def _():
        _k2_body(msg_hbm, nbr_hbm, smax_hbm, ia, gbA, gbB, obA, obB,
                 gsA, gsB, ssA, ssB, nrows, nbrow)


def _k2_body(msg_hbm, nbr_hbm, smax_hbm, ia, gbA, gbB, obA, obB,
             gsA, gsB, ssA, ssB, nrows, nbrow):
    pltpu.sync_copy(nbr_hbm.at[pl.ds(nbrow * K, RT0 * K)], ia)

    def gm(c, gb, sem):
        return pltpu.make_async_copy(
            msg_hbm.at[ia.at[pl.ds(c * K, G * K)]], gb, sem)

    def st(c, ob, sem):
        return pltpu.make_async_copy(
            ob, smax_hbm.at[pl.ds(nbrow + c, G)], sem)

    def compute(gb, ob):
        for u in range(G):
            accs = tuple(_mrow(gb, u * K))

            def body(k, accs, u=u):
                row = _mrow(gb, u * K + k)
                return tuple(jnp.maximum(a, r) for a, r in zip(accs, row))

            accs = lax.fori_loop(1, K, body, accs)
            for g in range(HG):
                ob[u, pl.ds(16 * g, 16)] = accs[g]

    gm(0, gbA, gsA).start()

    @pl.loop(0, nrows, step=2 * G)
    def _(c):
        gm(c + G, gbB, gsB).start()
        gm(c, gbA, gsA).wait()

        @pl.when(c >= 2 * G)
        def _():
            st(c - 2 * G, obA, ssA).wait()

        compute(gbA, obA)
        st(c, obA, ssA).start()

        @pl.when(c + 2 * G < nrows)
        def _():
            gm(c + 2 * G, gbA, gsA).start()

        gm(c + G, gbB, gsB).wait()

        @pl.when(c >= 2 * G)
        def _():
            st(c - G, obB, ssB).wait()

        compute(gbB, obB)
        st(c + G, obB, ssB).start()

    st(nrows - 2 * G, obA, ssA).wait()
    st(nrows - G, obB, ssB).wait()


@functools.partial(
    pl.kernel, mesh=_MESH, compiler_params=_SC_PARAMS,
    out_type=jax.ShapeDtypeStruct((NP, H), jnp.float32),
    scratch_types=[pltpu.VMEM((RT0 * K // 2,), jnp.int32),
                   pltpu.VMEM((RT0 * K // 2,), jnp.int32),
                   pltpu.VMEM((G * K, H), jnp.float32),
                   pltpu.VMEM((G * K, H), jnp.float32),
                   pltpu.VMEM((G * K, H), jnp.float32),
                   pltpu.VMEM((G * K, H), jnp.float32),
                   pltpu.VMEM((G, H), jnp.float32),
                   pltpu.VMEM((G, H), jnp.float32),
                   pltpu.SemaphoreType.DMA,
                   pltpu.SemaphoreType.DMA,
                   pltpu.SemaphoreType.DMA,
                   pltpu.SemaphoreType.DMA,
                   pltpu.SemaphoreType.DMA,
                   pltpu.SemaphoreType.DMA])
def _k3(msg_hbm, smax_hbm, nbr_hbm, d2_hbm, inv_hbm,
        ia, ib, gmA, gmB, gsA, gsB, obA, obB,
        smA, smB, sxA, sxB, ssA, ssB):
    nrows, nbrow0 = _span(RT0, RT1, C0ROWS)

    @pl.when(nrows > 0)
    def _():
        _k3_body(msg_hbm, smax_hbm, nbr_hbm, d2_hbm, inv_hbm, ia, ib,
                 gmA, gmB, gsA, gsB, obA, obB, smA, smB, sxA, sxB,
                 ssA, ssB, nrows, nbrow0)


def _k3_body(msg_hbm, smax_hbm, nbr_hbm, d2_hbm, inv_hbm, ia, ib,
             gmA, gmB, gsA, gsB, obA, obB, smA, smB, sxA, sxB,
             ssA, ssB, nrows, nbrow0):
    half = pl.multiple_of(nrows // 2, 8)

    def compute(gmb, gsb, ob):
        for u in range(G):
            accs = (jnp.zeros((16,), jnp.float32),) * HG

            def body(k, accs, u=u):
                ms = _mrow(gmb, u * K + k)
                out = []
                for g in range(HG):
                    s = gsb[u * K + k, pl.ds(16 * g, 16)]
                    out.append(accs[g] + jnp.exp(ms[g] - s))
                return tuple(out)

            accs = lax.fori_loop(0, K, body, accs)
            for g in range(HG):
                ob[u, pl.ds(16 * g, 16)] = 1.0 / (accs[g] + 1e-16)

    for h in range(2):
        nbrow = pl.multiple_of(nbrow0 + h * half, 8)
        pltpu.sync_copy(nbr_hbm.at[pl.ds(nbrow * K, RT0 * K // 2)], ia)
        pltpu.sync_copy(d2_hbm.at[pl.ds(nbrow * K, RT0 * K // 2)], ib)

        def gm(c, gb, sem):
            return pltpu.make_async_copy(
                msg_hbm.at[ia.at[pl.ds(c * K, G * K)]], gb, sem)

        def gs(c, gb, sem):
            return pltpu.make_async_copy(
                smax_hbm.at[ib.at[pl.ds(c * K, G * K)]], gb, sem)

        def st(c, ob, sem, nbrow=nbrow):
            return pltpu.make_async_copy(
                ob, inv_hbm.at[pl.ds(nbrow + c, G)], sem)

        gm(0, gmA, smA).start()
        gs(0, gsA, sxA).start()

        @pl.loop(0, half, step=2 * G)
        def _(c):
            gm(c + G, gmB, smB).start()
            gs(c + G, gsB, sxB).start()
            gm(c, gmA, smA).wait()
            gs(c, gsA, sxA).wait()

            @pl.when(c >= 2 * G)
            def _():
                st(c - 2 * G, obA, ssA).wait()

            compute(gmA, gsA, obA)
            st(c, obA, ssA).start()

            @pl.when(c + 2 * G < half)
            def _():
                gm(c + 2 * G, gmA, smA).start()
                gs(c + 2 * G, gsA, sxA).start()

            gm(c + G, gmB, smB).wait()
            gs(c + G, gsB, sxB).wait()

            @pl.when(c >= 2 * G)
            def _():
                st(c - G, obB, ssB).wait()

            compute(gmB, gsB, obB)
            st(c + G, obB, ssB).start()

        st(half - 2 * G, obA, ssA).wait()
        st(half - G, obB, ssB).wait()


@functools.partial(
    pl.kernel, mesh=_MESH, compiler_params=_SC_PARAMS,
    out_type=jax.ShapeDtypeStruct((NP, H), jnp.float32),
    scratch_types=[pltpu.VMEM((RT0 * K // 4,), jnp.int32),
                   pltpu.VMEM((RT0 * K // 4,), jnp.int32),
                   pltpu.VMEM((G * K, H), jnp.float32),
                   pltpu.VMEM((G * K, H), jnp.float32),
                   pltpu.VMEM((G * K, H), jnp.float32),
                   pltpu.VMEM((G * K, H), jnp.float32),
                   pltpu.VMEM((G * K, H), jnp.float32),
                   pltpu.VMEM((G * K, H), jnp.float32),
                   pltpu.VMEM((G, H), jnp.float32),
                   pltpu.VMEM((G, H), jnp.float32),
                   pltpu.SemaphoreType.DMA,
                   pltpu.SemaphoreType.DMA,
                   pltpu.SemaphoreType.DMA,
                   pltpu.SemaphoreType.DMA,
                   pltpu.SemaphoreType.DMA,
                   pltpu.SemaphoreType.DMA,
                   pltpu.SemaphoreType.DMA,
                   pltpu.SemaphoreType.DMA])
def _k4(msg_hbm, smax_hbm, inv_hbm, nbr_hbm, d2_hbm, res_hbm,
        ia, ib, gmA, gmB, gsA, gsB, giA, giB, obA, obB,
        smA, smB, sxA, sxB, siA, siB, ssA, ssB):
    nrows, nbrow0 = _span(RT0, RT1, C0ROWS)

    @pl.when(nrows > 0)
    def _():
        _k4_body(msg_hbm, smax_hbm, inv_hbm, nbr_hbm, d2_hbm, res_hbm,
                 ia, ib, gmA, gmB, gsA, gsB, giA, giB, obA, obB,
                 smA, smB, sxA, sxB, siA, siB, ssA, ssB, nrows, nbrow0)


def _k4_body(msg_hbm, smax_hbm, inv_hbm, nbr_hbm, d2_hbm, res_hbm,
             ia, ib, gmA, gmB, gsA, gsB, giA, giB, obA, obB,
             smA, smB, sxA, sxB, siA, siB, ssA, ssB, nrows, nbrow0):
    quart = pl.multiple_of(nrows // 4, 8)

    def compute(gmb, gsb, gib, ob):
        for u in range(G):
            accs = (jnp.zeros((16,), jnp.float32),) * HG

            def body(k, accs, u=u):
                ms = _mrow(gmb, u * K + k)
                out = []
                for g in range(HG):
                    s = gsb[u * K + k, pl.ds(16 * g, 16)]
                    iv = gib[u * K + k, pl.ds(16 * g, 16)]
                    out.append(accs[g] + ms[g] * jnp.exp(ms[g] - s) * iv)
                return tuple(out)

            accs = lax.fori_loop(0, K, body, accs)
            for g in range(HG):
                ob[u, pl.ds(16 * g, 16)] = accs[g]

    for h in range(4):
        nbrow = pl.multiple_of(nbrow0 + h * quart, 8)
        pltpu.sync_copy(nbr_hbm.at[pl.ds(nbrow * K, RT0 * K // 4)], ia)
        pltpu.sync_copy(d2_hbm.at[pl.ds(nbrow * K, RT0 * K // 4)], ib)

        def gm(c, gb, sem):
            return pltpu.make_async_copy(
                msg_hbm.at[ia.at[pl.ds(c * K, G * K)]], gb, sem)

        def gs(c, gb, sem):
            return pltpu.make_async_copy(
                smax_hbm.at[ib.at[pl.ds(c * K, G * K)]], gb, sem)

        def gi(c, gb, sem):
            return pltpu.make_async_copy(
                inv_hbm.at[ib.at[pl.ds(c * K, G * K)]], gb, sem)

        def st(c, ob, sem, nbrow=nbrow):
            return pltpu.make_async_copy(
                ob, res_hbm.at[pl.ds(nbrow + c, G)], sem)

        gm(0, gmA, smA).start()
        gs(0, gsA, sxA).start()
        gi(0, giA, siA).start()

        @pl.loop(0, quart, step=2 * G)
        def _(c):
            gm(c + G, gmB, smB).start()
            gs(c + G, gsB, sxB).start()
            gi(c + G, giB, siB).start()
            gm(c, gmA, smA).wait()
            gs(c, gsA, sxA).wait()
            gi(c, giA, siA).wait()

            @pl.when(c >= 2 * G)
            def _():
                st(c - 2 * G, obA, ssA).wait()

            compute(gmA, gsA, giA, obA)
            st(c, obA, ssA).start()

            @pl.when(c + 2 * G < quart)
            def _():
                gm(c + 2 * G, gmA, smA).start()
                gs(c + 2 * G, gsA, sxA).start()
                gi(c + 2 * G, giA, siA).start()

            gm(c + G, gmB, smB).wait()
            gs(c + G, gsB, sxB).wait()
            gi(c + G, giB, siB).wait()

            @pl.when(c >= 2 * G)
            def _():
                st(c - G, obB, ssB).wait()

            compute(gmB, gsB, giB, obB)
            st(c + G, obB, ssB).start()

        st(quart - 2 * G, obA, ssA).wait()
        st(quart - G, obB, ssB).wait()


def _tca_body(res_ref, x_ref, w1_ref, h1_ref, st_ref):
    i = pl.program_id(1)
    h = res_ref[...] + x_ref[...]
    h1 = lax.dot_general(h, w1_ref[0], (((1,), (1,)), ((), ())),
                         preferred_element_type=jnp.float32)
    h1_ref[0] = h1

    @pl.when(i == 0)
    def _():
        st_ref[...] = jnp.zeros_like(st_ref)

    st_ref[0, 0:1, :] += jnp.sum(h1, axis=0, keepdims=True)
    st_ref[0, 1:2, :] += jnp.sum(h1 * h1, axis=0, keepdims=True)


def _tca(res, x, w1):
    return pl.pallas_call(
        _tca_body,
        grid=(R, NBLKS),
        in_specs=[
            pl.BlockSpec((NB_BLK, H), lambda r, i: (r * NBLKS + i, 0)),
            pl.BlockSpec((NB_BLK, H), lambda r, i: (i, 0)),
            pl.BlockSpec((1, 2 * H, H), lambda r, i: (r, 0, 0)),
        ],
        out_specs=[
            pl.BlockSpec((1, NB_BLK, 2 * H), lambda r, i: (r, i, 0)),
            pl.BlockSpec((1, 8, 2 * H), lambda r, i: (r, 0, 0)),
        ],
        out_shape=[
            jax.ShapeDtypeStruct((R, N, 2 * H), jnp.float32),
            jax.ShapeDtypeStruct((R, 8, 2 * H), jnp.float32),
        ],
    )(res, x, w1)


def _tcb_body(h1_ref, st_ref, gam_ref, bet_ref, w2_ref, o_ref, *, leaky):
    nf = jnp.float32(N)
    acc = jnp.zeros((NB_BLK, H), jnp.float32)
    for r in range(R):
        mean = st_ref[r, 0:1, :] / nf
        var = st_ref[r, 1:2, :] / nf - mean * mean
        s = gam_ref[r:r + 1, :] * lax.rsqrt(var + 1e-5)
        b = bet_ref[r:r + 1, :] - mean * s
        h1n = jnp.maximum(h1_ref[r] * s + b, 0.0)
        acc = acc + lax.dot_general(h1n, w2_ref[r], (((1,), (1,)), ((), ())),
                                    preferred_element_type=jnp.float32)
    if leaky:
        acc = jnp.where(acc >= 0, acc, 0.01 * acc)
    o_ref[...] = acc


def _tcb(h1, st, gam, bet, w2, leaky):
    return pl.pallas_call(
        functools.partial(_tcb_body, leaky=leaky),
        grid=(NBLKS,),
        in_specs=[
            pl.BlockSpec((R, NB_BLK, 2 * H), lambda i: (0, i, 0)),
            pl.BlockSpec((R, 8, 2 * H), lambda i: (0, 0, 0)),
            pl.BlockSpec((R, 2 * H), lambda i: (0, 0)),
            pl.BlockSpec((R, 2 * H), lambda i: (0, 0)),
            pl.BlockSpec((R, H, 2 * H), lambda i: (0, 0, 0)),
        ],
        out_specs=pl.BlockSpec((NB_BLK, H), lambda i: (i, 0)),
        out_shape=jax.ShapeDtypeStruct((N, H), jnp.float32),
    )(h1, st, gam, bet, w2)


def kernel(x_hex, edge_inds, edge_attrs, nbrs, We, W1, gamma, beta, W2):
    roff = jnp.arange(R, dtype=jnp.int32)
    src_flat = jnp.pad(edge_inds[:, 0, :].reshape(R * E), (0, ET0))
    dstoff = (edge_inds[:, 1, :] + (roff * N)[:, None]).reshape(R * E)
    ea_flat = jnp.pad(edge_attrs.reshape(R * E), (0, ET0))
    nbr_flat = (nbrs + (roff * E)[:, None, None]).reshape(R * N, K)
    nbr_1d = jnp.pad(nbr_flat, ((0, NP - R * N), (0, 0))).reshape(NP * K)
    nbr_1d = jnp.pad(nbr_1d, (0, IPAD))
    We2 = We.reshape(L, R, H)

    d2_1d = _k0(nbr_1d, dstoff)
    x = x_hex
    for i in range(L):
        msg = _k1(x, src_flat, ea_flat, We2[i])
        smax = _k2(msg, nbr_1d)
        inv = _k3(msg, smax, nbr_1d, d2_1d)
        res = _k4(msg, smax, inv, nbr_1d, d2_1d)
        h1, st = _tca(res, x, W1[i])
        x = _tcb(h1, st, gamma[i], beta[i], W2[i], leaky=(i < L - 1))
    return x


# speedup vs baseline: 1.3365x; 1.3365x over previous
import functools

import jax
import jax.numpy as jnp
from jax import lax
from jax.experimental import pallas as pl
from jax.experimental.pallas import tpu as pltpu
from jax.experimental.pallas import tpu_sc as plsc

N = 10000
E = 160000
H = 128
R = 2
L = 2
K = 32
EPS = 1e-7

NW = 32
NP = 20480
G = 4
CE = 40
HG = H // 16

RT0, RT1 = 1120, 160
C0ROWS = 16 * RT0
ET0, ET1 = 10000, 10000
C0E = 16 * ET0
ST0, ST1 = 20480, 20480
C0S = 16 * ST0

IPAD = 40960

_MESH = plsc.VectorSubcoreMesh(core_axis_name="c", subcore_axis_name="s")
_SC_PARAMS = pltpu.CompilerParams(needs_layout_passes=False)

NB_BLK = 1000
NBLKS = N // NB_BLK


def _mrow(gb, row):
    return [gb[row, pl.ds(16 * g, 16)] for g in range(HG)]


def _span(t0, t1, c0total):
    c = lax.axis_index("c")
    s = lax.axis_index("s")
    cnt = pl.multiple_of(jnp.where(c == 0, t0, t1), 8)
    base = pl.multiple_of(jnp.where(c == 0, s * t0, c0total + s * t1), 8)
    return cnt, base


@functools.partial(
    pl.kernel, mesh=_MESH,
    out_type=jax.ShapeDtypeStruct((NP * K + IPAD,), jnp.int32),
    scratch_types=[pltpu.VMEM((128,), jnp.int32),
                   pltpu.VMEM((128,), jnp.int32)])
def _k0(nbr_hbm, dst_hbm, d2_hbm, idx_v, d2_v):
    cnt, base = _span(ST0, ST1, C0S)

    @pl.loop(0, cnt, step=128)
    def _(c):
        pltpu.sync_copy(nbr_hbm.at[pl.ds(base + c, 128)], idx_v)
        pltpu.sync_copy(dst_hbm.at[idx_v], d2_v)
        pltpu.sync_copy(d2_v, d2_hbm.at[pl.ds(base + c, 128)])


@functools.partial(
    pl.kernel, mesh=_MESH, compiler_params=_SC_PARAMS,
    out_type=jax.ShapeDtypeStruct((R * E, H), jnp.float32),
    scratch_types=[pltpu.VMEM((ET0,), jnp.int32),
                   pltpu.VMEM((ET0,), jnp.float32),
                   pltpu.VMEM((R, H), jnp.float32),
                   pltpu.VMEM((CE, H), jnp.float32),
                   pltpu.VMEM((CE, H), jnp.float32),
                   pltpu.VMEM((CE, H), jnp.float32),
                   pltpu.VMEM((CE, H), jnp.float32),
                   pltpu.SemaphoreType.DMA,
                   pltpu.SemaphoreType.DMA,
                   pltpu.SemaphoreType.DMA,
                   pltpu.SemaphoreType.DMA])
def _k1(x_hbm, src_hbm, ea_hbm, we_hbm, msg_hbm,
        ia, ab, wev, xbA, xbB, mbA, mbB, gsA, gsB, ssA, ssB):
    ept, base = _span(ET0, ET1, C0E)
    pltpu.sync_copy(we_hbm, wev)
    pltpu.sync_copy(src_hbm.at[pl.ds(base, ET0)], ia)
    pltpu.sync_copy(ea_hbm.at[pl.ds(base, ET0)], ab)
    we0 = [wev[0, pl.ds(16 * g, 16)] for g in range(HG)]
    we1 = [wev[1, pl.ds(16 * g, 16)] for g in range(HG)]

    def gx(c, xb, sem):
        return pltpu.make_async_copy(x_hbm.at[ia.at[pl.ds(c, CE)]], xb, sem)

    def st(c, mb, sem):
        return pltpu.make_async_copy(mb, msg_hbm.at[pl.ds(base + c, CE)], sem)

    def compute(c, xb, mb):
        rk = (base + c) >= E
        wegs = [jnp.where(rk, we1[g], we0[g]) for g in range(HG)]

        @pl.loop(0, CE)
        def _(j):
            a = plsc.load_gather(ab, [jnp.full((16,), c + j, jnp.int32)])
            ms = [jnp.maximum(xb[j, pl.ds(16 * g, 16)] + a * wegs[g], 0.0)
                  + EPS for g in range(HG)]
            for g in range(HG):
                mb[j, pl.ds(16 * g, 16)] = ms[g]

    gx(0, xbA, gsA).start()

    @pl.loop(0, ept, step=2 * CE)
    def _(c):
        gx(c + CE, xbB, gsB).start()
        gx(c, xbA, gsA).wait()

        @pl.when(c >= 2 * CE)
        def _():
            st(c - 2 * CE, mbA, ssA).wait()

        compute(c, xbA, mbA)
        st(c, mbA, ssA).start()

        @pl.when(c + 2 * CE < ept)
        def _():
            gx(c + 2 * CE, xbA, gsA).start()

        gx(c + CE, xbB, gsB).wait()

        @pl.when(c >= 2 * CE)
        def _():
            st(c - CE, mbB, ssB).wait()

        compute(c + CE, xbB, mbB)
        st(c + CE, mbB, ssB).start()

    st(ept - 2 * CE, mbA, ssA).wait()
    st(ept - CE, mbB, ssB).wait()


@functools.partial(
    pl.kernel, mesh=_MESH, compiler_params=_SC_PARAMS,
    out_type=jax.ShapeDtypeStruct((NP, H), jnp.float32),
    scratch_types=[pltpu.VMEM((RT0 * K,), jnp.int32),
                   pltpu.VMEM((G * K, H), jnp.float32),
                   pltpu.VMEM((G * K, H), jnp.float32),
                   pltpu.VMEM((G, H), jnp.float32),
                   pltpu.VMEM((G, H), jnp.float32),
                   pltpu.SemaphoreType.DMA,
                   pltpu.SemaphoreType.DMA,
                   pltpu.SemaphoreType.DMA,
                   pltpu.SemaphoreType.DMA])
def _k2(msg_hbm, nbr_hbm, smax_hbm,
        ia, gbA, gbB, obA, obB, gsA, gsB, ssA, ssB):
    nrows, nbrow = _span(RT0, RT1, C0ROWS)

    @pl.when(nrows > 0)
    def _():
        _k2_body(msg_hbm, nbr_hbm, smax_hbm, ia, gbA, gbB, obA, obB,
                 gsA, gsB, ssA, ssB, nrows, nbrow)


def _k2_body(msg_hbm, nbr_hbm, smax_hbm, ia, gbA, gbB, obA, obB,
             gsA, gsB, ssA, ssB, nrows, nbrow):
    pltpu.sync_copy(nbr_hbm.at[pl.ds(nbrow * K, RT0 * K)], ia)

    def gm(c, gb, sem):
        return pltpu.make_async_copy(
            msg_hbm.at[ia.at[pl.ds(c * K, G * K)]], gb, sem)

    def st(c, ob, sem):
        return pltpu.make_async_copy(
            ob, smax_hbm.at[pl.ds(nbrow + c, G)], sem)

    def compute(gb, ob):
        for u in range(G):
            accs = tuple(_mrow(gb, u * K))

            def body(k, accs, u=u):
                row = _mrow(gb, u * K + k)
                return tuple(jnp.maximum(a, r) for a, r in zip(accs, row))

            accs = lax.fori_loop(1, K, body, accs)
            for g in range(HG):
                ob[u, pl.ds(16 * g, 16)] = accs[g]

    gm(0, gbA, gsA).start()

    @pl.loop(0, nrows, step=2 * G)
    def _(c):
        gm(c + G, gbB, gsB).start()
        gm(c, gbA, gsA).wait()

        @pl.when(c >= 2 * G)
        def _():
            st(c - 2 * G, obA, ssA).wait()

        compute(gbA, obA)
        st(c, obA, ssA).start()

        @pl.when(c + 2 * G < nrows)
        def _():
            gm(c + 2 * G, gbA, gsA).start()

        gm(c + G, gbB, gsB).wait()

        @pl.when(c >= 2 * G)
        def _():
            st(c - G, obB, ssB).wait()

        compute(gbB, obB)
        st(c + G, obB, ssB).start()

    st(nrows - 2 * G, obA, ssA).wait()
    st(nrows - G, obB, ssB).wait()


@functools.partial(
    pl.kernel, mesh=_MESH, compiler_params=_SC_PARAMS,
    out_type=jax.ShapeDtypeStruct((NP, H), jnp.float32),
    scratch_types=[pltpu.VMEM((RT0 * K // 2,), jnp.int32),
                   pltpu.VMEM((RT0 * K // 2,), jnp.int32),
                   pltpu.VMEM((G * K, H), jnp.float32),
                   pltpu.VMEM((G * K, H), jnp.float32),
                   pltpu.VMEM((G * K, H), jnp.float32),
                   pltpu.VMEM((G * K, H), jnp.float32),
                   pltpu.VMEM((G, H), jnp.float32),
                   pltpu.VMEM((G, H), jnp.float32),
                   pltpu.SemaphoreType.DMA,
                   pltpu.SemaphoreType.DMA,
                   pltpu.SemaphoreType.DMA,
                   pltpu.SemaphoreType.DMA,
                   pltpu.SemaphoreType.DMA,
                   pltpu.SemaphoreType.DMA])
def _k3(msg_hbm, smax_hbm, nbr_hbm, d2_hbm, inv_hbm,
        ia, ib, gmA, gmB, gsA, gsB, obA, obB,
        smA, smB, sxA, sxB, ssA, ssB):
    nrows, nbrow0 = _span(RT0, RT1, C0ROWS)

    @pl.when(nrows > 0)
    def _():
        _k3_body(msg_hbm, smax_hbm, nbr_hbm, d2_hbm, inv_hbm, ia, ib,
                 gmA, gmB, gsA, gsB, obA, obB, smA, smB, sxA, sxB,
                 ssA, ssB, nrows, nbrow0)


def _k3_body(msg_hbm, smax_hbm, nbr_hbm, d2_hbm, inv_hbm, ia, ib,
             gmA, gmB, gsA, gsB, obA, obB, smA, smB, sxA, sxB,
             ssA, ssB, nrows, nbrow0):
    half = pl.multiple_of(nrows // 2, 8)

    def compute(gmb, gsb, ob):
        for u in range(G):
            accs = (jnp.zeros((16,), jnp.float32),) * HG

            def body(k, accs, u=u):
                ms = _mrow(gmb, u * K + k)
                out = []
                for g in range(HG):
                    s = gsb[u * K + k, pl.ds(16 * g, 16)]
                    out.append(accs[g] + jnp.exp(ms[g] - s))
                return tuple(out)

            accs = lax.fori_loop(0, K, body, accs)
            for g in range(HG):
                ob[u, pl.ds(16 * g, 16)] = 1.0 / (accs[g] + 1e-16)

    for h in range(2):
        nbrow = pl.multiple_of(nbrow0 + h * half, 8)
        pltpu.sync_copy(nbr_hbm.at[pl.ds(nbrow * K, RT0 * K // 2)], ia)
        pltpu.sync_copy(d2_hbm.at[pl.ds(nbrow * K, RT0 * K // 2)], ib)

        def gm(c, gb, sem):
            return pltpu.make_async_copy(
                msg_hbm.at[ia.at[pl.ds(c * K, G * K)]], gb, sem)

        def gs(c, gb, sem):
            return pltpu.make_async_copy(
                smax_hbm.at[ib.at[pl.ds(c * K, G * K)]], gb, sem)

        def st(c, ob, sem, nbrow=nbrow):
            return pltpu.make_async_copy(
                ob, inv_hbm.at[pl.ds(nbrow + c, G)], sem)

        gm(0, gmA, smA).start()
        gs(0, gsA, sxA).start()

        @pl.loop(0, half, step=2 * G)
        def _(c):
            gm(c + G, gmB, smB).start()
            gs(c + G, gsB, sxB).start()
            gm(c, gmA, smA).wait()
            gs(c, gsA, sxA).wait()

            @pl.when(c >= 2 * G)
            def _():
                st(c - 2 * G, obA, ssA).wait()

            compute(gmA, gsA, obA)
            st(c, obA, ssA).start()

            @pl.when(c + 2 * G < half)
            def _():
                gm(c + 2 * G, gmA, smA).start()
                gs(c + 2 * G, gsA, sxA).start()

            gm(c + G, gmB, smB).wait()
            gs(c + G, gsB, sxB).wait()

            @pl.when(c >= 2 * G)
            def _():
                st(c - G, obB, ssB).wait()

            compute(gmB, gsB, obB)
            st(c + G, obB, ssB).start()

        st(half - 2 * G, obA, ssA).wait()
        st(half - G, obB, ssB).wait()


@functools.partial(
    pl.kernel, mesh=_MESH, compiler_params=_SC_PARAMS,
    out_type=jax.ShapeDtypeStruct((NP, H), jnp.float32),
    scratch_types=[pltpu.VMEM((RT0 * K // 4,), jnp.int32),
                   pltpu.VMEM((RT0 * K // 4,), jnp.int32),
                   pltpu.VMEM((G * K, H), jnp.float32),
                   pltpu.VMEM((G * K, H), jnp.float32),
                   pltpu.VMEM((G * K, H), jnp.float32),
                   pltpu.VMEM((G * K, H), jnp.float32),
                   pltpu.VMEM((G * K, H), jnp.float32),
                   pltpu.VMEM((G * K, H), jnp.float32),
                   pltpu.VMEM((G, H), jnp.float32),
                   pltpu.VMEM((G, H), jnp.float32),
                   pltpu.SemaphoreType.DMA,
                   pltpu.SemaphoreType.DMA,
                   pltpu.SemaphoreType.DMA,
                   pltpu.SemaphoreType.DMA,
                   pltpu.SemaphoreType.DMA,
                   pltpu.SemaphoreType.DMA,
                   pltpu.SemaphoreType.DMA,
                   pltpu.SemaphoreType.DMA])
def _k4(msg_hbm, smax_hbm, inv_hbm, nbr_hbm, d2_hbm, res_hbm,
        ia, ib, gmA, gmB, gsA, gsB, giA, giB, obA, obB,
        smA, smB, sxA, sxB, siA, siB, ssA, ssB):
    nrows, nbrow0 = _span(RT0, RT1, C0ROWS)

    @pl.when(nrows > 0)
    def _():
        _k4_body(msg_hbm, smax_hbm, inv_hbm, nbr_hbm, d2_hbm, res_hbm,
                 ia, ib, gmA, gmB, gsA, gsB, giA, giB, obA, obB,
                 smA, smB, sxA, sxB, siA, siB, ssA, ssB, nrows, nbrow0)


def _k4_body(msg_hbm, smax_hbm, inv_hbm, nbr_hbm, d2_hbm, res_hbm,
             ia, ib, gmA, gmB, gsA, gsB, giA, giB, obA, obB,
             smA, smB, sxA, sxB, siA, siB, ssA, ssB, nrows, nbrow0):
    quart = pl.multiple_of(nrows // 4, 8)

    def compute(gmb, gsb, gib, ob):
        for u in range(G):
            accs = (jnp.zeros((16,), jnp.float32),) * HG

            def body(k, accs, u=u):
                ms = _mrow(gmb, u * K + k)
                out = []
                for g in range(HG):
                    s = gsb[u * K + k, pl.ds(16 * g, 16)]
                    iv = gib[u * K + k, pl.ds(16 * g, 16)]
                    out.append(accs[g] + ms[g] * jnp.exp(ms[g] - s) * iv)
                return tuple(out)

            accs = lax.fori_loop(0, K, body, accs)
            for g in range(HG):
                ob[u, pl.ds(16 * g, 16)] = accs[g]

    for h in range(4):
        nbrow = pl.multiple_of(nbrow0 + h * quart, 8)
        pltpu.sync_copy(nbr_hbm.at[pl.ds(nbrow * K, RT0 * K // 4)], ia)
        pltpu.sync_copy(d2_hbm.at[pl.ds(nbrow * K, RT0 * K // 4)], ib)

        def gm(c, gb, sem):
            return pltpu.make_async_copy(
                msg_hbm.at[ia.at[pl.ds(c * K, G * K)]], gb, sem)

        def gs(c, gb, sem):
            return pltpu.make_async_copy(
                smax_hbm.at[ib.at[pl.ds(c * K, G * K)]], gb, sem)

        def gi(c, gb, sem):
            return pltpu.make_async_copy(
                inv_hbm.at[ib.at[pl.ds(c * K, G * K)]], gb, sem)

        def st(c, ob, sem, nbrow=nbrow):
            return pltpu.make_async_copy(
                ob, res_hbm.at[pl.ds(nbrow + c, G)], sem)

        gm(0, gmA, smA).start()
        gs(0, gsA, sxA).start()
        gi(0, giA, siA).start()

        @pl.loop(0, quart, step=2 * G)
        def _(c):
            gm(c + G, gmB, smB).start()
            gs(c + G, gsB, sxB).start()
            gi(c + G, giB, siB).start()
            gm(c, gmA, smA).wait()
            gs(c, gsA, sxA).wait()
            gi(c, giA, siA).wait()

            @pl.when(c >= 2 * G)
            def _():
                st(c - 2 * G, obA, ssA).wait()

            compute(gmA, gsA, giA, obA)
            st(c, obA, ssA).start()

            @pl.when(c + 2 * G < quart)
            def _():
                gm(c + 2 * G, gmA, smA).start()
                gs(c + 2 * G, gsA, sxA).start()
                gi(c + 2 * G, giA, siA).start()

            gm(c + G, gmB, smB).wait()
            gs(c + G, gsB, sxB).wait()
            gi(c + G, giB, siB).wait()

            @pl.when(c >= 2 * G)
            def _():
                st(c - G, obB, ssB).wait()

            compute(gmB, gsB, giB, obB)
            st(c + G, obB, ssB).start()

        st(quart - 2 * G, obA, ssA).wait()
        st(quart - G, obB, ssB).wait()


def _tca_body(res_ref, x_ref, w1_ref, h1_ref, st_ref):
    i = pl.program_id(1)
    h = res_ref[...] + x_ref[...]
    h1 = lax.dot_general(h, w1_ref[0], (((1,), (1,)), ((), ())),
                         preferred_element_type=jnp.float32)
    h1_ref[0] = h1

    @pl.when(i == 0)
    def _():
        st_ref[...] = jnp.zeros_like(st_ref)

    st_ref[0, 0:1, :] += jnp.sum(h1, axis=0, keepdims=True)
    st_ref[0, 1:2, :] += jnp.sum(h1 * h1, axis=0, keepdims=True)


def _tca(res, x, w1):
    return pl.pallas_call(
        _tca_body,
        grid=(R, NBLKS),
        in_specs=[
            pl.BlockSpec((NB_BLK, H), lambda r, i: (r * NBLKS + i, 0)),
            pl.BlockSpec((NB_BLK, H), lambda r, i: (i, 0)),
            pl.BlockSpec((1, 2 * H, H), lambda r, i: (r, 0, 0)),
        ],
        out_specs=[
            pl.BlockSpec((1, NB_BLK, 2 * H), lambda r, i: (r, i, 0)),
            pl.BlockSpec((1, 8, 2 * H), lambda r, i: (r, 0, 0)),
        ],
        out_shape=[
            jax.ShapeDtypeStruct((R, N, 2 * H), jnp.float32),
            jax.ShapeDtypeStruct((R, 8, 2 * H), jnp.float32),
        ],
    )(res, x, w1)


def _tcb_body(h1_ref, st_ref, gam_ref, bet_ref, w2_ref, o_ref, *, leaky):
    nf = jnp.float32(N)
    acc = jnp.zeros((NB_BLK, H), jnp.float32)
    for r in range(R):
        mean = st_ref[r, 0:1, :] / nf
        var = st_ref[r, 1:2, :] / nf - mean * mean
        s = gam_ref[r:r + 1, :] * lax.rsqrt(var + 1e-5)
        b = bet_ref[r:r + 1, :] - mean * s
        h1n = jnp.maximum(h1_ref[r] * s + b, 0.0)
        acc = acc + lax.dot_general(h1n, w2_ref[r], (((1,), (1,)), ((), ())),
                                    preferred_element_type=jnp.float32)
    if leaky:
        acc = jnp.where(acc >= 0, acc, 0.01 * acc)
    o_ref[...] = acc


def _tcb(h1, st, gam, bet, w2, leaky):
    return pl.pallas_call(
        functools.partial(_tcb_body, leaky=leaky),
        grid=(NBLKS,),
        in_specs=[
            pl.BlockSpec((R, NB_BLK, 2 * H), lambda i: (0, i, 0)),
            pl.BlockSpec((R, 8, 2 * H), lambda i: (0, 0, 0)),
            pl.BlockSpec((R, 2 * H), lambda i: (0, 0)),
            pl.BlockSpec((R, 2 * H), lambda i: (0, 0)),
            pl.BlockSpec((R, H, 2 * H), lambda i: (0, 0, 0)),
        ],
        out_specs=pl.BlockSpec((NB_BLK, H), lambda i: (i, 0)),
        out_shape=jax.ShapeDtypeStruct((N, H), jnp.float32),
    )(h1, st, gam, bet, w2)


def kernel(x_hex, edge_inds, edge_attrs, nbrs, We, W1, gamma, beta, W2):
    roff = jnp.arange(R, dtype=jnp.int32)
    src_flat = jnp.pad(edge_inds[:, 0, :].reshape(R * E), (0, ET0))
    dstoff = (edge_inds[:, 1, :] + (roff * N)[:, None]).reshape(R * E)
    ea_flat = jnp.pad(edge_attrs.reshape(R * E), (0, ET0))
    nbr_flat = (nbrs + (roff * E)[:, None, None]).reshape(R * N, K)
    nbr_1d = jnp.pad(nbr_flat, ((0, NP - R * N), (0, 0))).reshape(NP * K)
    nbr_1d = jnp.pad(nbr_1d, (0, IPAD))
    We2 = We.reshape(L, R, H)

    d2_1d = _k0(nbr_1d, dstoff)
    x = x_hex
    for i in range(L):
        msg = _k1(x, src_flat, ea_flat, We2[i])
        smax = _k2(msg, nbr_1d)
        inv = _k3(msg, smax, nbr_1d, d2_1d)
        res = _k4(msg, smax, inv, nbr_1d, d2_1d)
        h1, st = _tca(res, x, W1[i])
        x = _tcb(h1, st, gamma[i], beta[i], W2[i], leaky=(i < L - 1))
    return x
